# baseline (speedup 1.0000x reference)
import functools

import jax
import jax.numpy as jnp
from jax import lax
from jax.experimental import pallas as pl
from jax.experimental.pallas import tpu as pltpu
from jax.experimental.pallas import tpu_sc as plsc

BATCH = 16384
EMBED_DIM = 64

_NC = 2
_NS = 16
_NW = _NC * _NS
_B_PER_W = BATCH // _NW
_CHUNK = 128
_N_CHUNKS = _B_PER_W // _CHUNK

_mesh = plsc.VectorSubcoreMesh(core_axis_name="c", subcore_axis_name="s")


@functools.partial(
    pl.kernel,
    mesh=_mesh,
    compiler_params=pltpu.CompilerParams(use_tc_tiling_on_sc=False),
    out_type=jax.ShapeDtypeStruct((BATCH, EMBED_DIM), jnp.float32),
    scratch_types=[
        pltpu.VMEM((_N_CHUNKS, _CHUNK), jnp.int32),
        pltpu.VMEM((_B_PER_W, EMBED_DIM), jnp.float32),
        pltpu.SemaphoreType.DMA,
    ],
)
def _gather(idx_hbm, table_hbm, out_hbm, idx_v, rows_v, sem):
    wid = lax.axis_index("s") * _NC + lax.axis_index("c")
    pltpu.sync_copy(idx_hbm.at[wid], idx_v)
    copies = [
        pltpu.async_copy(
            table_hbm.at[idx_v.at[j]],
            rows_v.at[pl.ds(j * _CHUNK, _CHUNK)],
            sem,
        )
        for j in range(_N_CHUNKS)
    ]
    for c in copies:
        c.wait()
    pltpu.sync_copy(rows_v, out_hbm.at[pl.ds(wid * _B_PER_W, _B_PER_W)])


def kernel(nodes, table):
    idx = nodes.astype(jnp.int32).reshape(_NW, _N_CHUNKS, _CHUNK)
    return _gather(idx, table)


# speedup vs baseline: 1.8322x; 1.8322x over previous
import functools

import jax
import jax.numpy as jnp
from jax import lax
from jax.experimental import pallas as pl
from jax.experimental.pallas import tpu as pltpu
from jax.experimental.pallas import tpu_sc as plsc

NUM_NODES = 1000000
BATCH = 16384
EMBED_DIM = 64

_NC = 2
_NS = 16
_NW = _NC * _NS
_B_PER_W = BATCH // _NW
_CHUNK = 16
_N_CHUNKS = _B_PER_W // _CHUNK

_mesh = plsc.VectorSubcoreMesh(core_axis_name="c", subcore_axis_name="s")


@functools.partial(
    pl.kernel,
    mesh=_mesh,
    compiler_params=pltpu.CompilerParams(needs_layout_passes=False),
    out_type=jax.ShapeDtypeStruct((EMBED_DIM * BATCH,), jnp.float32),
    scratch_types=[
        pltpu.VMEM((_B_PER_W,), jnp.int32),
        pltpu.VMEM((8, 8, 128), jnp.float32),
        pltpu.VMEM((8, 8, 128), jnp.float32),
        pltpu.VMEM((EMBED_DIM * _B_PER_W,), jnp.float32),
        pltpu.SemaphoreType.DMA,
        pltpu.SemaphoreType.DMA,
        pltpu.SemaphoreType.DMA,
    ],
)
def _gather_t(idx_hbm, tab4_hbm, out_hbm, idx_v, buf0, buf1, out_stage,
              sem0, sem1, sem_out):
    wid = lax.axis_index("s") * _NC + lax.axis_index("c")
    base = wid * _B_PER_W
    pltpu.sync_copy(idx_hbm.at[pl.ds(base, _B_PER_W)], idx_v)

    iota = lax.iota(jnp.int32, 16)
    zeros = jnp.zeros((16,), jnp.int32)
    bufs = (buf0, buf1)
    sems = (sem0, sem1)

    def body(c, carry):
        vec = idx_v[pl.ds(c * _CHUNK, _CHUNK)]
        copies = []
        for k in range(_CHUNK):
            r = vec[k]
            r128 = pl.multiple_of(r - lax.bitwise_and(r, 127), 128)
            copies.append(pltpu.async_copy(
                tab4_hbm.at[:, :, pl.ds(r128, 128)],
                bufs[k % 2],
                sems[k % 2],
            ))
            if k >= 1:
                copies[k - 1].wait()
                rp = vec[k - 1]
                lane_vec = zeros + lax.bitwise_and(rp, 127)
                b_local = c * _CHUNK + (k - 1)
                for t in range(4):
                    i_vec = 2 * t + iota // 8
                    s_vec = iota % 8
                    vals = plsc.load_gather(bufs[(k - 1) % 2],
                                            [i_vec, s_vec, lane_vec])
                    pos = (16 * t + iota) * _B_PER_W + b_local
                    plsc.store_scatter(out_stage, [pos], vals)
        copies[_CHUNK - 1].wait()
        rp = vec[_CHUNK - 1]
        lane_vec = zeros + lax.bitwise_and(rp, 127)
        b_local = c * _CHUNK + (_CHUNK - 1)
        for t in range(4):
            i_vec = 2 * t + iota // 8
            s_vec = iota % 8
            vals = plsc.load_gather(bufs[(_CHUNK - 1) % 2],
                                    [i_vec, s_vec, lane_vec])
            pos = (16 * t + iota) * _B_PER_W + b_local
            plsc.store_scatter(out_stage, [pos], vals)
        return carry

    lax.fori_loop(0, _N_CHUNKS, body, 0)

    out_copies = []
    for d in range(EMBED_DIM):
        out_copies.append(pltpu.async_copy(
            out_stage.at[pl.ds(d * _B_PER_W, _B_PER_W)],
            out_hbm.at[pl.ds(d * BATCH + base, _B_PER_W)],
            sem_out,
        ))
    for cp in out_copies:
        cp.wait()


def kernel(nodes, table):
    idx = nodes.astype(jnp.int32)
    tab4 = table.T.reshape(8, 8, NUM_NODES)
    out1 = _gather_t(idx, tab4)
    return out1.reshape(EMBED_DIM, BATCH).T


# speedup vs baseline: 2.3410x; 1.2777x over previous
import functools

import jax
import jax.numpy as jnp
from jax import lax
from jax.experimental import pallas as pl
from jax.experimental.pallas import tpu as pltpu
from jax.experimental.pallas import tpu_sc as plsc

NUM_NODES = 1000000
BATCH = 16384
EMBED_DIM = 64

_NC = 2
_NS = 16
_NW = _NC * _NS
_B_PER_W = BATCH // _NW
_CHUNK = 16
_N_CHUNKS = _B_PER_W // _CHUNK

_mesh = plsc.VectorSubcoreMesh(core_axis_name="c", subcore_axis_name="s")


@functools.partial(
    pl.kernel,
    mesh=_mesh,
    compiler_params=pltpu.CompilerParams(needs_layout_passes=False),
    out_type=jax.ShapeDtypeStruct((EMBED_DIM * BATCH,), jnp.float32),
    scratch_types=[
        pltpu.VMEM((_B_PER_W,), jnp.int32),
        pltpu.VMEM((8, 8, 128), jnp.float32),
        pltpu.VMEM((8, 8, 128), jnp.float32),
        pltpu.VMEM((8, 8, 128), jnp.float32),
        pltpu.VMEM((8, 8, 128), jnp.float32),
        pltpu.VMEM((EMBED_DIM * _B_PER_W,), jnp.float32),
        pltpu.SemaphoreType.DMA,
        pltpu.SemaphoreType.DMA,
        pltpu.SemaphoreType.DMA,
        pltpu.SemaphoreType.DMA,
        pltpu.SemaphoreType.DMA,
    ],
)
def _gather_t(idx_hbm, tab4_hbm, out_hbm, idx_v, buf0, buf1, buf2, buf3,
              out_stage, sem0, sem1, sem2, sem3, sem_out):
    wid = lax.axis_index("s") * _NC + lax.axis_index("c")
    base = wid * _B_PER_W
    pltpu.sync_copy(idx_hbm.at[pl.ds(base, _B_PER_W)], idx_v)

    iota = lax.iota(jnp.int32, 16)
    zeros = jnp.zeros((16,), jnp.int32)
    bufs = (buf0, buf1, buf2, buf3)
    sems = (sem0, sem1, sem2, sem3)
    _LAG = len(bufs) - 1

    def extract(buf, r, b_local):
        lane_vec = zeros + lax.bitwise_and(r, 127)
        for t in range(4):
            i_vec = 2 * t + iota // 8
            s_vec = iota % 8
            vals = plsc.load_gather(buf, [i_vec, s_vec, lane_vec])
            pos = (16 * t + iota) * _B_PER_W + b_local
            plsc.store_scatter(out_stage, [pos], vals)

    def body(c, carry):
        vec = idx_v[pl.ds(c * _CHUNK, _CHUNK)]
        copies = []
        for k in range(_CHUNK):
            r = vec[k]
            r128 = pl.multiple_of(r - lax.bitwise_and(r, 127), 128)
            copies.append(pltpu.async_copy(
                tab4_hbm.at[:, :, pl.ds(r128, 128)],
                bufs[k % len(bufs)],
                sems[k % len(bufs)],
            ))
            if k >= _LAG:
                kp = k - _LAG
                copies[kp].wait()
                extract(bufs[kp % len(bufs)], vec[kp], c * _CHUNK + kp)
        for kp in range(_CHUNK - _LAG, _CHUNK):
            copies[kp].wait()
            extract(bufs[kp % len(bufs)], vec[kp], c * _CHUNK + kp)
        return carry

    lax.fori_loop(0, _N_CHUNKS, body, 0)

    out_copies = []
    for d in range(EMBED_DIM):
        out_copies.append(pltpu.async_copy(
            out_stage.at[pl.ds(d * _B_PER_W, _B_PER_W)],
            out_hbm.at[pl.ds(d * BATCH + base, _B_PER_W)],
            sem_out,
        ))
    for cp in out_copies:
        cp.wait()


def kernel(nodes, table):
    idx = nodes.astype(jnp.int32)
    tab4 = table.T.reshape(8, 8, NUM_NODES)
    out1 = _gather_t(idx, tab4)
    return out1.reshape(EMBED_DIM, BATCH).T


# speedup vs baseline: 2.5955x; 1.1087x over previous
import functools

import jax
import jax.numpy as jnp
from jax import lax
from jax.experimental import pallas as pl
from jax.experimental.pallas import tpu as pltpu
from jax.experimental.pallas import tpu_sc as plsc

NUM_NODES = 1000000
BATCH = 16384
EMBED_DIM = 64

_NC = 2
_NS = 16
_NW = _NC * _NS
_B_PER_W = BATCH // _NW
_CHUNK = 16
_N_CHUNKS = _B_PER_W // _CHUNK

_mesh = plsc.VectorSubcoreMesh(core_axis_name="c", subcore_axis_name="s")


@functools.partial(
    pl.kernel,
    mesh=_mesh,
    compiler_params=pltpu.CompilerParams(needs_layout_passes=False),
    out_type=jax.ShapeDtypeStruct((EMBED_DIM * BATCH,), jnp.float32),
    scratch_types=[
        pltpu.VMEM((_B_PER_W,), jnp.int32),
        pltpu.VMEM((8, 8, 128), jnp.float32),
        pltpu.VMEM((8, 8, 128), jnp.float32),
        pltpu.VMEM((8, 8, 128), jnp.float32),
        pltpu.VMEM((8, 8, 128), jnp.float32),
        pltpu.VMEM((8, 8, 128), jnp.float32),
        pltpu.VMEM((8, 8, 128), jnp.float32),
        pltpu.VMEM((8, 8, 128), jnp.float32),
        pltpu.VMEM((8, 8, 128), jnp.float32),
        pltpu.VMEM((EMBED_DIM * _B_PER_W,), jnp.float32),
        pltpu.SemaphoreType.DMA,
        pltpu.SemaphoreType.DMA,
        pltpu.SemaphoreType.DMA,
        pltpu.SemaphoreType.DMA,
        pltpu.SemaphoreType.DMA,
        pltpu.SemaphoreType.DMA,
        pltpu.SemaphoreType.DMA,
        pltpu.SemaphoreType.DMA,
        pltpu.SemaphoreType.DMA,
    ],
)
def _gather_t(idx_hbm, tab4_hbm, out_hbm, idx_v, buf0, buf1, buf2, buf3,
              buf4, buf5, buf6, buf7, out_stage, sem0, sem1, sem2, sem3,
              sem4, sem5, sem6, sem7, sem_out):
    wid = lax.axis_index("s") * _NC + lax.axis_index("c")
    base = wid * _B_PER_W
    pltpu.sync_copy(idx_hbm.at[pl.ds(base, _B_PER_W)], idx_v)

    iota = lax.iota(jnp.int32, 16)
    zeros = jnp.zeros((16,), jnp.int32)
    bufs = (buf0, buf1, buf2, buf3, buf4, buf5, buf6, buf7)
    sems = (sem0, sem1, sem2, sem3, sem4, sem5, sem6, sem7)
    _LAG = len(bufs) - 1

    def extract(buf, r, b_local):
        lane_vec = zeros + lax.bitwise_and(r, 127)
        for t in range(4):
            i_vec = 2 * t + iota // 8
            s_vec = iota % 8
            vals = plsc.load_gather(buf, [i_vec, s_vec, lane_vec])
            pos = (16 * t + iota) * _B_PER_W + b_local
            plsc.store_scatter(out_stage, [pos], vals)

    def body(c, carry):
        vec = idx_v[pl.ds(c * _CHUNK, _CHUNK)]
        copies = []
        for k in range(_CHUNK):
            r = vec[k]
            r128 = pl.multiple_of(r - lax.bitwise_and(r, 127), 128)
            copies.append(pltpu.async_copy(
                tab4_hbm.at[:, :, pl.ds(r128, 128)],
                bufs[k % len(bufs)],
                sems[k % len(bufs)],
            ))
            if k >= _LAG:
                kp = k - _LAG
                copies[kp].wait()
                extract(bufs[kp % len(bufs)], vec[kp], c * _CHUNK + kp)
        for kp in range(_CHUNK - _LAG, _CHUNK):
            copies[kp].wait()
            extract(bufs[kp % len(bufs)], vec[kp], c * _CHUNK + kp)
        return carry

    lax.fori_loop(0, _N_CHUNKS, body, 0)

    out_copies = []
    for d in range(EMBED_DIM):
        out_copies.append(pltpu.async_copy(
            out_stage.at[pl.ds(d * _B_PER_W, _B_PER_W)],
            out_hbm.at[pl.ds(d * BATCH + base, _B_PER_W)],
            sem_out,
        ))
    for cp in out_copies:
        cp.wait()


def kernel(nodes, table):
    idx = nodes.astype(jnp.int32)
    tab4 = table.T.reshape(8, 8, NUM_NODES)
    out1 = _gather_t(idx, tab4)
    return out1.reshape(EMBED_DIM, BATCH).T


# speedup vs baseline: 2.7576x; 1.0625x over previous
import functools

import jax
import jax.numpy as jnp
from jax import lax
from jax.experimental import pallas as pl
from jax.experimental.pallas import tpu as pltpu
from jax.experimental.pallas import tpu_sc as plsc

NUM_NODES = 1000000
BATCH = 16384
EMBED_DIM = 64

_NC = 2
_NS = 16
_NW = _NC * _NS
_B_PER_W = BATCH // _NW
_CHUNK = 32
_N_CHUNKS = _B_PER_W // _CHUNK

_mesh = plsc.VectorSubcoreMesh(core_axis_name="c", subcore_axis_name="s")


@functools.partial(
    pl.kernel,
    mesh=_mesh,
    compiler_params=pltpu.CompilerParams(needs_layout_passes=False),
    out_type=jax.ShapeDtypeStruct((EMBED_DIM * BATCH,), jnp.float32),
    scratch_types=[
        pltpu.VMEM((_B_PER_W,), jnp.int32),
        pltpu.VMEM((8, 8, 128), jnp.float32),
        pltpu.VMEM((8, 8, 128), jnp.float32),
        pltpu.VMEM((8, 8, 128), jnp.float32),
        pltpu.VMEM((8, 8, 128), jnp.float32),
        pltpu.VMEM((8, 8, 128), jnp.float32),
        pltpu.VMEM((8, 8, 128), jnp.float32),
        pltpu.VMEM((8, 8, 128), jnp.float32),
        pltpu.VMEM((8, 8, 128), jnp.float32),
        pltpu.VMEM((EMBED_DIM * _B_PER_W,), jnp.float32),
        pltpu.SemaphoreType.DMA,
        pltpu.SemaphoreType.DMA,
        pltpu.SemaphoreType.DMA,
        pltpu.SemaphoreType.DMA,
        pltpu.SemaphoreType.DMA,
        pltpu.SemaphoreType.DMA,
        pltpu.SemaphoreType.DMA,
        pltpu.SemaphoreType.DMA,
        pltpu.SemaphoreType.DMA,
    ],
)
def _gather_t(idx_hbm, tab4_hbm, out_hbm, idx_v, buf0, buf1, buf2, buf3,
              buf4, buf5, buf6, buf7, out_stage, sem0, sem1, sem2, sem3,
              sem4, sem5, sem6, sem7, sem_out):
    wid = lax.axis_index("s") * _NC + lax.axis_index("c")
    base = wid * _B_PER_W
    pltpu.sync_copy(idx_hbm.at[pl.ds(base, _B_PER_W)], idx_v)

    iota = lax.iota(jnp.int32, 16)
    zeros = jnp.zeros((16,), jnp.int32)
    bufs = (buf0, buf1, buf2, buf3, buf4, buf5, buf6, buf7)
    sems = (sem0, sem1, sem2, sem3, sem4, sem5, sem6, sem7)
    _LAG = len(bufs) - 1

    def extract(buf, r, b_local):
        lane_vec = zeros + lax.bitwise_and(r, 127)
        for t in range(4):
            i_vec = 2 * t + iota // 8
            s_vec = iota % 8
            vals = plsc.load_gather(buf, [i_vec, s_vec, lane_vec])
            pos = (16 * t + iota) * _B_PER_W + b_local
            plsc.store_scatter(out_stage, [pos], vals)

    def body(c, carry):
        vec_lo = idx_v[pl.ds(c * _CHUNK, 16)]
        vec_hi = idx_v[pl.ds(c * _CHUNK + 16, 16)]

        def vec(k):
            return vec_lo[k] if k < 16 else vec_hi[k - 16]

        copies = []
        for k in range(_CHUNK):
            r = vec(k)
            r128 = pl.multiple_of(r - lax.bitwise_and(r, 127), 128)
            copies.append(pltpu.async_copy(
                tab4_hbm.at[:, :, pl.ds(r128, 128)],
                bufs[k % len(bufs)],
                sems[k % len(bufs)],
            ))
            if k >= _LAG:
                kp = k - _LAG
                copies[kp].wait()
                extract(bufs[kp % len(bufs)], vec(kp), c * _CHUNK + kp)
        for kp in range(_CHUNK - _LAG, _CHUNK):
            copies[kp].wait()
            extract(bufs[kp % len(bufs)], vec(kp), c * _CHUNK + kp)
        return carry

    lax.fori_loop(0, _N_CHUNKS, body, 0)

    out_copies = []
    for d in range(EMBED_DIM):
        out_copies.append(pltpu.async_copy(
            out_stage.at[pl.ds(d * _B_PER_W, _B_PER_W)],
            out_hbm.at[pl.ds(d * BATCH + base, _B_PER_W)],
            sem_out,
        ))
    for cp in out_copies:
        cp.wait()


def kernel(nodes, table):
    idx = nodes.astype(jnp.int32)
    tab4 = table.T.reshape(8, 8, NUM_NODES)
    out1 = _gather_t(idx, tab4)
    return out1.reshape(EMBED_DIM, BATCH).T
